# baseline (speedup 1.0000x reference)
import functools

import jax
import jax.numpy as jnp
from jax import lax
from jax.experimental import pallas as pl
from jax.experimental.pallas import tpu as pltpu
from jax.experimental.pallas import tpu_sc as plsc

N_NODES = 10000
N_EDGES = 320000
D = 128
L = 16
NC = 2
NS = 16
NW = NC * NS
CHUNK = 128
TRIPS = 80
NCHUNK = TRIPS * NW
E_PAD = NCHUNK * CHUNK
QUADS = TRIPS // 4
ROWS_LO = 624
ROWS_HI = N_NODES - 15 * ROWS_LO
ZROWS = 16


def _sc_partials(x, packed, pw):
    mesh = plsc.VectorSubcoreMesh(
        core_axis_name="c", subcore_axis_name="s", num_cores=NC, num_subcores=NS
    )

    @functools.partial(
        pl.kernel,
        out_type=jax.ShapeDtypeStruct((NC, N_NODES, D), jnp.float32),
        mesh=mesh,
        scratch_types=[
            pltpu.VMEM_SHARED((N_NODES, D), jnp.float32),
            pltpu.VMEM((4, 2, CHUNK), jnp.int32),
            pltpu.VMEM((4, 1, CHUNK), jnp.float32),
            pltpu.VMEM((2, CHUNK, D), jnp.float32),
            pltpu.VMEM((ZROWS, D), jnp.float32),
            pltpu.SemaphoreType.DMA,
            pltpu.SemaphoreType.DMA,
            pltpu.SemaphoreType.DMA,
            pltpu.SemaphoreType.DMA,
            pltpu.SemaphoreType.DMA,
            pltpu.SemaphoreType.DMA,
            pltpu.SemaphoreType.DMA,
            pltpu.SemaphoreType.DMA,
        ],
    )
    def k(x_hbm, pk_hbm, pw_hbm, parts_hbm, acc, pbuf, pwbuf, rows, zbuf,
          si0, si1, si2, si3, sg0, sg1, ss0, ss1):
        cid = lax.axis_index("c")
        sid = lax.axis_index("s")
        wid = sid * NC + cid
        base_row = sid * ROWS_LO
        sem_i = (si0, si1, si2, si3)
        sem_g = (sg0, sg1)
        sem_s = (ss0, ss1)

        zeros = jnp.zeros((L,), jnp.float32)
        for r in range(ZROWS):
            for j in range(D // L):
                zbuf[r, pl.ds(j * L, L)] = zeros

        def zcopy(kk, _):
            pltpu.sync_copy(zbuf, acc.at[pl.ds(base_row + kk * ZROWS, ZROWS)])
            return 0

        n_owned = jnp.where(sid == NS - 1, ROWS_HI, ROWS_LO)
        lax.fori_loop(0, n_owned // ZROWS, zcopy, 0)
        plsc.subcore_barrier()

        def issue_idx(trip, q):
            c = trip * NW + wid
            pltpu.async_copy(pk_hbm.at[c], pbuf.at[q], sem_i[q])
            pltpu.async_copy(pw_hbm.at[c], pwbuf.at[q], sem_i[q])

        def wait_idx(q):
            pltpu.make_async_copy(pk_hbm.at[0], pbuf.at[q], sem_i[q]).wait()
            pltpu.make_async_copy(pw_hbm.at[0], pwbuf.at[q], sem_i[q]).wait()

        def issue_gath(q, p):
            pltpu.async_copy(x_hbm.at[pbuf.at[q, 0]], rows.at[p], sem_g[p])

        def wait_gath(q, p):
            pltpu.make_async_copy(x_hbm.at[pbuf.at[q, 0]], rows.at[p],
                                  sem_g[p]).wait()

        def issue_scat(q, p):
            pltpu.async_copy(rows.at[p], acc.at[pbuf.at[q, 1]], sem_s[p],
                             add=True)

        def wait_scat(q, p):
            pltpu.make_async_copy(rows.at[p], acc.at[pbuf.at[q, 1]],
                                  sem_s[p]).wait()

        def scale(q, p):
            def body(g, _):
                wg = pwbuf[q, 0, pl.ds(g * L, L)]
                for ee in range(L):
                    e = g * L + ee
                    ws = wg[ee]
                    for cc in range(D // L):
                        sl = pl.ds(cc * L, L)
                        rows[p, e, sl] = rows[p, e, sl] * ws
                return 0

            lax.fori_loop(0, CHUNK // L, body, 0)

        issue_idx(0, 0)
        issue_idx(1, 1)
        wait_idx(0)
        issue_gath(0, 0)

        def quad_body(i4, _):
            t0 = 4 * i4
            for kq in range(4):
                t = t0 + kq
                p = kq % 2
                q = kq
                qn = (kq + 1) % 4
                qnn = (kq + 2) % 4

                wait_gath(q, p)

                if kq == 0:
                    @pl.when(t > 0)
                    def _():
                        wait_scat(3, 1 - p)
                else:
                    wait_scat(kq - 1, 1 - p)

                def launch_next():
                    wait_idx(qn)
                    issue_gath(qn, 1 - p)

                if kq == 3:
                    @pl.when(t + 1 < TRIPS)
                    def _():
                        launch_next()
                else:
                    launch_next()

                scale(q, p)
                issue_scat(q, p)

                def prefetch():
                    issue_idx(t + 2, qnn)

                if kq >= 2:
                    @pl.when(t + 2 < TRIPS)
                    def _():
                        prefetch()
                else:
                    prefetch()
            return 0

        lax.fori_loop(0, QUADS, quad_body, 0)
        wait_scat(3, 1)
        plsc.subcore_barrier()

        @pl.when(sid < NS - 1)
        def _():
            pltpu.sync_copy(
                acc.at[pl.ds(base_row, ROWS_LO)],
                parts_hbm.at[cid, pl.ds(base_row, ROWS_LO)],
            )

        @pl.when(sid == NS - 1)
        def _():
            pltpu.sync_copy(
                acc.at[pl.ds(15 * ROWS_LO, ROWS_HI)],
                parts_hbm.at[cid, pl.ds(15 * ROWS_LO, ROWS_HI)],
            )

    return k(x, packed, pw)


def _combine(p0, p1):
    def body(a_ref, b_ref, o_ref):
        o_ref[...] = jnp.clip(a_ref[...] + b_ref[...], -20.0, 20.0)

    blk = 1000
    spec = pl.BlockSpec((blk, D), lambda i: (i, 0))
    return pl.pallas_call(
        body,
        grid=(N_NODES // blk,),
        in_specs=[spec, spec],
        out_specs=spec,
        out_shape=jax.ShapeDtypeStruct((N_NODES, D), jnp.float32),
    )(p0, p1)


def kernel(t, x, edge_index, edge_weight):
    pad = E_PAD - N_EDGES
    src = jnp.concatenate([edge_index[1], jnp.zeros((pad,), jnp.int32)])
    dst = jnp.concatenate([edge_index[0], jnp.zeros((pad,), jnp.int32)])
    w = jnp.concatenate([edge_weight, jnp.zeros((pad,), jnp.float32)])
    packed = jnp.concatenate(
        [src.reshape(NCHUNK, 1, CHUNK), dst.reshape(NCHUNK, 1, CHUNK)],
        axis=1,
    )
    pw = w.reshape(NCHUNK, 1, CHUNK)
    parts = _sc_partials(x, packed, pw)
    return _combine(parts[0], parts[1])


# speedup vs baseline: 2.1523x; 2.1523x over previous
import functools

import jax
import jax.numpy as jnp
from jax import lax
from jax.experimental import pallas as pl
from jax.experimental.pallas import tpu as pltpu
from jax.experimental.pallas import tpu_sc as plsc

N_NODES = 10000
N_EDGES = 320000
D = 128
L = 16
NC = 2
NS = 16
NW = NC * NS
SUB = 128
CHUNK = 256
NSUB = CHUNK // SUB
NCHUNK = N_EDGES // CHUNK
TRIPS = (NCHUNK + NW - 1) // NW
PAIRS = TRIPS // 2
ROWS_LO = 624
ROWS_HI = N_NODES - 15 * ROWS_LO
ZROWS = 16


def _sc_partials(x, packed, pw):
    mesh = plsc.VectorSubcoreMesh(
        core_axis_name="c", subcore_axis_name="s", num_cores=NC, num_subcores=NS
    )

    @functools.partial(
        pl.kernel,
        out_type=jax.ShapeDtypeStruct((NC, N_NODES, D), jnp.float32),
        mesh=mesh,
        scratch_types=[
            pltpu.VMEM_SHARED((N_NODES, D), jnp.float32),
            pltpu.VMEM((2, 2 * NSUB, SUB), jnp.int32),
            pltpu.VMEM((2, NSUB, SUB), jnp.float32),
            pltpu.VMEM((CHUNK, D), jnp.float32),
            pltpu.VMEM((ZROWS, D), jnp.float32),
            pltpu.SemaphoreType.DMA,
            pltpu.SemaphoreType.DMA,
            pltpu.SemaphoreType.DMA,
            pltpu.SemaphoreType.DMA,
            pltpu.SemaphoreType.DMA,
        ],
    )
    def k(x_hbm, pk_hbm, pw_hbm, parts_hbm, acc, pbuf, pwbuf, rows, zbuf,
          sem_g0, sem_g1, sem_i0, sem_i1, sem_sc):
        cid = lax.axis_index("c")
        sid = lax.axis_index("s")
        wid = sid * NC + cid
        base_row = sid * ROWS_LO

        zeros = jnp.zeros((L,), jnp.float32)
        for r in range(ZROWS):
            for j in range(D // L):
                zbuf[r, pl.ds(j * L, L)] = zeros

        def zcopy(kk, _):
            pltpu.sync_copy(zbuf, acc.at[pl.ds(base_row + kk * ZROWS, ZROWS)])
            return 0

        n_owned = jnp.where(sid == NS - 1, ROWS_HI, ROWS_LO)
        lax.fori_loop(0, n_owned // ZROWS, zcopy, 0)
        plsc.subcore_barrier()

        my_trips = jnp.where(wid < NCHUNK - (TRIPS - 1) * NW, TRIPS, TRIPS - 1)
        sems = (sem_i0, sem_i1)

        def issue_idx(trip, pb):
            c = trip * NW + wid
            pltpu.async_copy(pk_hbm.at[c], pbuf.at[pb], sems[pb])
            pltpu.async_copy(pw_hbm.at[c], pwbuf.at[pb], sems[pb])

        def wait_idx(pb):
            pltpu.make_async_copy(pk_hbm.at[0], pbuf.at[pb], sems[pb]).wait()
            pltpu.make_async_copy(pw_hbm.at[0], pwbuf.at[pb], sems[pb]).wait()

        def do_trip(trip, pb):
            wait_idx(pb)
            sem_g = (sem_g0, sem_g1)
            gs = [
                pltpu.async_copy(x_hbm.at[pbuf.at[pb, j]],
                                 rows.at[pl.ds(j * SUB, SUB)], sem_g[j])
                for j in range(NSUB)
            ]

            scs = []
            for j in range(NSUB):
                gs[j].wait()

                def scale(g, _):
                    wg = pwbuf[pb, j, pl.ds(g * L, L)]
                    for ee in range(L):
                        e = j * SUB + g * L + ee
                        ws = wg[ee]
                        for q in range(D // L):
                            sl = pl.ds(q * L, L)
                            rows[e, sl] = rows[e, sl] * ws
                    return 0

                lax.fori_loop(0, SUB // L, scale, 0)
                scs.append(
                    pltpu.async_copy(rows.at[pl.ds(j * SUB, SUB)],
                                     acc.at[pbuf.at[pb, NSUB + j]], sem_sc,
                                     add=True)
                )

            for sdesc in scs:
                sdesc.wait()

            @pl.when(trip + 2 < my_trips)
            def _():
                issue_idx(trip + 2, pb)

        issue_idx(0, 0)

        @pl.when(1 < my_trips)
        def _():
            issue_idx(1, 1)

        def pair_body(i2, _):
            t = 2 * i2

            @pl.when(t < my_trips)
            def _():
                do_trip(t, 0)

            @pl.when(t + 1 < my_trips)
            def _():
                do_trip(t + 1, 1)

            return 0

        lax.fori_loop(0, PAIRS, pair_body, 0)
        plsc.subcore_barrier()

        @pl.when(sid < NS - 1)
        def _():
            pltpu.sync_copy(
                acc.at[pl.ds(base_row, ROWS_LO)],
                parts_hbm.at[cid, pl.ds(base_row, ROWS_LO)],
            )

        @pl.when(sid == NS - 1)
        def _():
            pltpu.sync_copy(
                acc.at[pl.ds(15 * ROWS_LO, ROWS_HI)],
                parts_hbm.at[cid, pl.ds(15 * ROWS_LO, ROWS_HI)],
            )

    return k(x, packed, pw)


def _combine(p0, p1):
    def body(a_ref, b_ref, o_ref):
        o_ref[...] = jnp.clip(a_ref[...] + b_ref[...], -20.0, 20.0)

    blk = 1000
    spec = pl.BlockSpec((blk, D), lambda i: (i, 0))
    return pl.pallas_call(
        body,
        grid=(N_NODES // blk,),
        in_specs=[spec, spec],
        out_specs=spec,
        out_shape=jax.ShapeDtypeStruct((N_NODES, D), jnp.float32),
    )(p0, p1)


def kernel(t, x, edge_index, edge_weight):
    src = edge_index[1].reshape(NCHUNK, NSUB, SUB)
    dst = edge_index[0].reshape(NCHUNK, NSUB, SUB)
    pw = edge_weight.reshape(NCHUNK, NSUB, SUB)
    packed = jnp.concatenate([src, dst], axis=1)
    parts = _sc_partials(x, packed, pw)
    return _combine(parts[0], parts[1])
